# Initial kernel scaffold; baseline (speedup 1.0000x reference)
import jax
import jax.numpy as jnp
from jax.experimental import pallas as pl


def kernel(query, k, enc_W1, enc_b1, enc_g, enc_beta, enc_W2, enc_b2, dec_W1, dec_b1, dec_g, dec_beta, dec_W2, dec_b2, concepts):
    raise NotImplementedError("write your pallas kernel here")


# speedup vs baseline: 2.7137x; 2.7137x over previous
import jax
import jax.numpy as jnp
from jax import lax
from jax.experimental import pallas as pl
from jax.experimental.pallas import tpu as pltpu

B = 64
D = 128
K = 16
T = 8000
L = 40
NEG = -3.0


def _layer_norm(x, g, b, eps=1e-5):
    mu = jnp.mean(x, axis=-1, keepdims=True)
    var = jnp.mean((x - mu) * (x - mu), axis=-1, keepdims=True)
    return (x - mu) / jnp.sqrt(var + eps) * g + b


def _mlp(x, W1, b1, g, beta, W2, b2):
    h = lax.dot_general(x, W1, (((1,), (1,)), ((), ())),
                        preferred_element_type=jnp.float32) + b1
    h = _layer_norm(h, g, beta)
    h = h * 0.5 * (1.0 + lax.erf(h * 0.7071067811865476))
    return lax.dot_general(h, W2, (((1,), (1,)), ((), ())),
                           preferred_element_type=jnp.float32) + b2


def _encode_body(q_ref, w1_ref, b1_ref, g_ref, be_ref, w2_ref, b2_ref,
                 e_ref, qn_ref):
    e = _mlp(q_ref[...], w1_ref[...], b1_ref[...], g_ref[...], be_ref[...],
             w2_ref[...], b2_ref[...])
    e_ref[...] = e
    qn_ref[...] = jnp.sqrt(jnp.sum(e * e, axis=1, keepdims=True))


def _encode(query, W1, b1, g, beta, W2, b2):
    spec = pl.BlockSpec((B, D), lambda: (0, 0))
    wspec = pl.BlockSpec((D, D), lambda: (0, 0))
    vspec = pl.BlockSpec((1, D), lambda: (0, 0))
    return pl.pallas_call(
        _encode_body,
        out_shape=[jax.ShapeDtypeStruct((B, D), jnp.float32),
                   jax.ShapeDtypeStruct((B, 1), jnp.float32)],
        in_specs=[spec, wspec, vspec, vspec, vspec, wspec, vspec],
        out_specs=[spec, pl.BlockSpec((B, 1), lambda: (0, 0))],
    )(query, W1, b1, g, beta, W2, b2)


def _scan_body(c_ref, e_ref, bm_ref):
    tile = c_ref[...]
    cn = jnp.sqrt(jnp.sum(tile * tile, axis=1, keepdims=True))
    dots = lax.dot_general(tile, e_ref[...], (((1,), (1,)), ((), ())),
                           preferred_element_type=jnp.float32)
    sims = dots / jnp.maximum(cn, 1e-8)
    bm_ref[...] = jnp.max(sims.reshape(T // L, L, B), axis=1)


def _scan(concepts, e):
    n = concepts.shape[0]
    ntiles = n // T
    nb = n // L
    return pl.pallas_call(
        _scan_body,
        grid=(ntiles,),
        out_shape=jax.ShapeDtypeStruct((nb, B), jnp.float32),
        in_specs=[
            pl.BlockSpec((T, D), lambda t: (t, 0)),
            pl.BlockSpec((B, D), lambda t: (0, 0)),
        ],
        out_specs=pl.BlockSpec((T // L, B), lambda t: (t, 0)),
        compiler_params=pltpu.CompilerParams(
            dimension_semantics=("arbitrary",)),
    )(concepts, e)


def _topk_blocks_body(bm_ref, blk_ref):
    v = bm_ref[...]
    nb = v.shape[0]
    riota = lax.broadcasted_iota(jnp.int32, v.shape, 0)
    for i in range(K):
        m = jnp.max(v, axis=0, keepdims=True)
        rows = jnp.where(v == m, riota, nb)
        r = jnp.min(rows, axis=0)
        blk_ref[i, :] = r
        v = jnp.where(riota == r[None, :], NEG, v)


def _topk_blocks(blockmax):
    nb = blockmax.shape[0]
    return pl.pallas_call(
        _topk_blocks_body,
        out_shape=jax.ShapeDtypeStruct((K, B), jnp.int32),
        in_specs=[pl.BlockSpec((nb, B), lambda: (0, 0))],
        out_specs=pl.BlockSpec((K, B), lambda: (0, 0)),
    )(blockmax)


def _select_body(blk_ref, c_ref, e_ref, qn_ref, w1_ref, b1_ref, g_ref,
                 be_ref, w2_ref, b2_ref, dec_ref, sim_ref, cand_ref):
    b = pl.program_id(0)
    j = pl.program_id(1)
    cand_ref[pl.ds(j * L, L), :] = c_ref[...]

    @pl.when(j == K - 1)
    def _():
        cand = cand_ref[...]
        cn = jnp.sqrt(jnp.sum(cand * cand, axis=1, keepdims=True))
        dots_all = lax.dot_general(cand, e_ref[...], (((1,), (1,)), ((), ())),
                                   preferred_element_type=jnp.float32)
        qsel = lax.broadcasted_iota(jnp.int32, (1, B), 1) == b
        dots = jnp.sum(jnp.where(qsel, dots_all, 0.0), axis=1,
                       keepdims=True)
        qn_b = qn_ref[pl.ds(b, 1), :]
        v = dots / jnp.maximum(cn * qn_b, 1e-8)
        riota = lax.broadcasted_iota(jnp.int32, v.shape, 0)
        oh_row = lax.broadcasted_iota(jnp.int32, (K, K * L), 0)
        oh_lane = lax.broadcasted_iota(jnp.int32, (K, K * L), 1)
        kiota = lax.broadcasted_iota(jnp.int32, (1, K), 1)
        oh = jnp.zeros((K, K * L), jnp.float32)
        simrow = jnp.zeros((1, K), jnp.float32)
        for i in range(K):
            m = jnp.max(v, axis=0, keepdims=True)
            rows = jnp.where(v == m, riota, K * L)
            sel = jnp.min(rows, axis=0, keepdims=True)
            oh = jnp.where((oh_row == i) & (oh_lane == sel), 1.0, oh)
            simrow = jnp.where(kiota == i, m, simrow)
            v = jnp.where(riota == sel, NEG, v)
        retrieved = lax.dot_general(oh, cand, (((1,), (0,)), ((), ())),
                                    preferred_element_type=jnp.float32,
                                    precision=lax.Precision.HIGHEST)
        decoded = _mlp(retrieved, w1_ref[...], b1_ref[...], g_ref[...],
                       be_ref[...], w2_ref[...], b2_ref[...])
        dec_ref[...] = decoded.reshape(1, K, D)
        sim_ref[...] = simrow.reshape(1, 1, K)


def _select(blk_flat, concepts, e, qn, W1, b1, g, beta, W2, b2):
    wspec = pl.BlockSpec((D, D), lambda b, j, s: (0, 0))
    vspec = pl.BlockSpec((1, D), lambda b, j, s: (0, 0))
    grid_spec = pltpu.PrefetchScalarGridSpec(
        num_scalar_prefetch=1,
        grid=(B, K),
        in_specs=[
            pl.BlockSpec((L, D), lambda b, j, s: (s[b * K + j], 0)),
            pl.BlockSpec((B, D), lambda b, j, s: (0, 0)),
            pl.BlockSpec((B, 1), lambda b, j, s: (0, 0)),
            wspec, vspec, vspec, vspec, wspec, vspec,
        ],
        out_specs=[
            pl.BlockSpec((1, K, D), lambda b, j, s: (b, 0, 0)),
            pl.BlockSpec((1, 1, K), lambda b, j, s: (b, 0, 0)),
        ],
        scratch_shapes=[pltpu.VMEM((K * L, D), jnp.float32)],
    )
    return pl.pallas_call(
        _select_body,
        grid_spec=grid_spec,
        out_shape=[
            jax.ShapeDtypeStruct((B, K, D), jnp.float32),
            jax.ShapeDtypeStruct((B, 1, K), jnp.float32),
        ],
        compiler_params=pltpu.CompilerParams(
            dimension_semantics=("arbitrary", "arbitrary")),
    )(blk_flat, concepts, e, qn, W1, b1, g, beta, W2, b2)


def kernel(query, k, enc_W1, enc_b1, enc_g, enc_beta, enc_W2, enc_b2,
           dec_W1, dec_b1, dec_g, dec_beta, dec_W2, dec_b2, concepts):
    r = lambda v: v.reshape(1, D)
    e, qn = _encode(query, enc_W1, r(enc_b1), r(enc_g), r(enc_beta),
                    enc_W2, r(enc_b2))
    blockmax = _scan(concepts, e)
    blk = _topk_blocks(blockmax)
    blk_flat = blk.T.reshape(-1)
    decoded, sims = _select(blk_flat, concepts, e, qn, dec_W1, r(dec_b1),
                            r(dec_g), r(dec_beta), dec_W2, r(dec_b2))
    return decoded, sims.reshape(B, K)


# speedup vs baseline: 2.7138x; 1.0000x over previous
import jax
import jax.numpy as jnp
from jax import lax
from jax.experimental import pallas as pl
from jax.experimental.pallas import tpu as pltpu

B = 64
D = 128
K = 16
T = 8000
L = 40
NEG = -3.0


def _layer_norm(x, g, b, eps=1e-5):
    mu = jnp.mean(x, axis=-1, keepdims=True)
    var = jnp.mean((x - mu) * (x - mu), axis=-1, keepdims=True)
    return (x - mu) / jnp.sqrt(var + eps) * g + b


def _mlp(x, W1, b1, g, beta, W2, b2):
    h = lax.dot_general(x, W1, (((1,), (1,)), ((), ())),
                        preferred_element_type=jnp.float32) + b1
    h = _layer_norm(h, g, beta)
    h = h * 0.5 * (1.0 + lax.erf(h * 0.7071067811865476))
    return lax.dot_general(h, W2, (((1,), (1,)), ((), ())),
                           preferred_element_type=jnp.float32) + b2


def _encode_body(q_ref, w1_ref, b1_ref, g_ref, be_ref, w2_ref, b2_ref,
                 e_ref, qn_ref):
    e = _mlp(q_ref[...], w1_ref[...], b1_ref[...], g_ref[...], be_ref[...],
             w2_ref[...], b2_ref[...])
    e_ref[...] = e
    qn_ref[...] = jnp.sqrt(jnp.sum(e * e, axis=1, keepdims=True))


def _encode(query, W1, b1, g, beta, W2, b2):
    spec = pl.BlockSpec((B, D), lambda: (0, 0))
    wspec = pl.BlockSpec((D, D), lambda: (0, 0))
    vspec = pl.BlockSpec((1, D), lambda: (0, 0))
    return pl.pallas_call(
        _encode_body,
        out_shape=[jax.ShapeDtypeStruct((B, D), jnp.float32),
                   jax.ShapeDtypeStruct((B, 1), jnp.float32)],
        in_specs=[spec, wspec, vspec, vspec, vspec, wspec, vspec],
        out_specs=[spec, pl.BlockSpec((B, 1), lambda: (0, 0))],
    )(query, W1, b1, g, beta, W2, b2)


def _scan_body(c_ref, e_ref, bm_ref):
    tile = c_ref[...]
    cn = jnp.sqrt(jnp.sum(tile * tile, axis=1, keepdims=True))
    dots = lax.dot_general(tile, e_ref[...], (((1,), (1,)), ((), ())),
                           preferred_element_type=jnp.float32)
    sims = dots * (1.0 / jnp.maximum(cn, 1e-8))
    bm_ref[...] = jnp.max(sims.reshape(T // L, L, B), axis=1)


def _scan(concepts, e):
    n = concepts.shape[0]
    ntiles = n // T
    nb = n // L
    return pl.pallas_call(
        _scan_body,
        grid=(ntiles,),
        out_shape=jax.ShapeDtypeStruct((nb, B), jnp.float32),
        in_specs=[
            pl.BlockSpec((T, D), lambda t: (t, 0)),
            pl.BlockSpec((B, D), lambda t: (0, 0)),
        ],
        out_specs=pl.BlockSpec((T // L, B), lambda t: (t, 0)),
        compiler_params=pltpu.CompilerParams(
            dimension_semantics=("arbitrary",)),
    )(concepts, e)


def _topk_blocks_body(bm_ref, blk_ref):
    v = bm_ref[...]
    nb = v.shape[0]
    riota = lax.broadcasted_iota(jnp.int32, v.shape, 0)
    for i in range(K):
        m = jnp.max(v, axis=0, keepdims=True)
        rows = jnp.where(v == m, riota, nb)
        r = jnp.min(rows, axis=0)
        blk_ref[i, :] = r
        v = jnp.where(riota == r[None, :], NEG, v)


def _topk_blocks(blockmax):
    nb = blockmax.shape[0]
    return pl.pallas_call(
        _topk_blocks_body,
        out_shape=jax.ShapeDtypeStruct((K, B), jnp.int32),
        in_specs=[pl.BlockSpec((nb, B), lambda: (0, 0))],
        out_specs=pl.BlockSpec((K, B), lambda: (0, 0)),
    )(blockmax)


def _select_body(blk_ref, c_ref, e_ref, qn_ref, w1_ref, b1_ref, g_ref,
                 be_ref, w2_ref, b2_ref, dec_ref, sim_ref, cand_ref):
    b = pl.program_id(0)
    j = pl.program_id(1)
    cand_ref[pl.ds(j * L, L), :] = c_ref[...]

    @pl.when(j == K - 1)
    def _():
        cand = cand_ref[...]
        cn = jnp.sqrt(jnp.sum(cand * cand, axis=1, keepdims=True))
        dots_all = lax.dot_general(cand, e_ref[...], (((1,), (1,)), ((), ())),
                                   preferred_element_type=jnp.float32)
        qsel = lax.broadcasted_iota(jnp.int32, (1, B), 1) == b
        dots = jnp.sum(jnp.where(qsel, dots_all, 0.0), axis=1,
                       keepdims=True)
        qn_b = qn_ref[pl.ds(b, 1), :]
        v = dots / jnp.maximum(cn * qn_b, 1e-8)
        riota = lax.broadcasted_iota(jnp.int32, v.shape, 0)
        oh_row = lax.broadcasted_iota(jnp.int32, (K, K * L), 0)
        oh_lane = lax.broadcasted_iota(jnp.int32, (K, K * L), 1)
        kiota = lax.broadcasted_iota(jnp.int32, (1, K), 1)
        oh = jnp.zeros((K, K * L), jnp.float32)
        simrow = jnp.zeros((1, K), jnp.float32)
        for i in range(K):
            m = jnp.max(v, axis=0, keepdims=True)
            rows = jnp.where(v == m, riota, K * L)
            sel = jnp.min(rows, axis=0, keepdims=True)
            oh = jnp.where((oh_row == i) & (oh_lane == sel), 1.0, oh)
            simrow = jnp.where(kiota == i, m, simrow)
            v = jnp.where(riota == sel, NEG, v)
        retrieved = lax.dot_general(oh, cand, (((1,), (0,)), ((), ())),
                                    preferred_element_type=jnp.float32,
                                    precision=lax.Precision.HIGHEST)
        decoded = _mlp(retrieved, w1_ref[...], b1_ref[...], g_ref[...],
                       be_ref[...], w2_ref[...], b2_ref[...])
        dec_ref[...] = decoded.reshape(1, K, D)
        sim_ref[...] = simrow.reshape(1, 1, K)


def _select(blk_flat, concepts, e, qn, W1, b1, g, beta, W2, b2):
    wspec = pl.BlockSpec((D, D), lambda b, j, s: (0, 0))
    vspec = pl.BlockSpec((1, D), lambda b, j, s: (0, 0))
    grid_spec = pltpu.PrefetchScalarGridSpec(
        num_scalar_prefetch=1,
        grid=(B, K),
        in_specs=[
            pl.BlockSpec((L, D), lambda b, j, s: (s[b * K + j], 0)),
            pl.BlockSpec((B, D), lambda b, j, s: (0, 0)),
            pl.BlockSpec((B, 1), lambda b, j, s: (0, 0)),
            wspec, vspec, vspec, vspec, wspec, vspec,
        ],
        out_specs=[
            pl.BlockSpec((1, K, D), lambda b, j, s: (b, 0, 0)),
            pl.BlockSpec((1, 1, K), lambda b, j, s: (b, 0, 0)),
        ],
        scratch_shapes=[pltpu.VMEM((K * L, D), jnp.float32)],
    )
    return pl.pallas_call(
        _select_body,
        grid_spec=grid_spec,
        out_shape=[
            jax.ShapeDtypeStruct((B, K, D), jnp.float32),
            jax.ShapeDtypeStruct((B, 1, K), jnp.float32),
        ],
        compiler_params=pltpu.CompilerParams(
            dimension_semantics=("arbitrary", "arbitrary")),
    )(blk_flat, concepts, e, qn, W1, b1, g, beta, W2, b2)


def kernel(query, k, enc_W1, enc_b1, enc_g, enc_beta, enc_W2, enc_b2,
           dec_W1, dec_b1, dec_g, dec_beta, dec_W2, dec_b2, concepts):
    r = lambda v: v.reshape(1, D)
    e, qn = _encode(query, enc_W1, r(enc_b1), r(enc_g), r(enc_beta),
                    enc_W2, r(enc_b2))
    blockmax = _scan(concepts, e)
    blk = _topk_blocks(blockmax)
    blk_flat = blk.T.reshape(-1)
    decoded, sims = _select(blk_flat, concepts, e, qn, dec_W1, r(dec_b1),
                            r(dec_g), r(dec_beta), dec_W2, r(dec_b2))
    return decoded, sims.reshape(B, K)
